# baseline (speedup 1.0000x reference)
import functools

import jax
import jax.numpy as jnp
from jax import lax
from jax.experimental import pallas as pl
from jax.experimental.pallas import tpu as pltpu
from jax.experimental.pallas import tpu_sc as plsc

D = 256
E = 160000
NW = 32
NE = 5120
EP = NW * NE
CH = 128
NCHUNK = NE // CH
NG = CH // 16


def _body(h_hbm, u_hbm, v_hbm, out_hbm, u_idx, v_idx, ur, vr, sc, sem_u, sem_v):
    wid = lax.axis_index("s") * 2 + lax.axis_index("c")
    base = wid * NE
    pltpu.sync_copy(u_hbm.at[pl.ds(base, NE)], u_idx)
    pltpu.sync_copy(v_hbm.at[pl.ds(base, NE)], v_idx)

    lane = lax.iota(jnp.int32, 16)
    rows = [lane + g * 16 for g in range(NG)]

    def chunk_body(ci, carry):
        cu = pltpu.async_copy(
            h_hbm.at[u_idx.at[pl.ds(ci * CH, CH)]], ur, sem_u)
        cv = pltpu.async_copy(
            h_hbm.at[v_idx.at[pl.ds(ci * CH, CH)]], vr, sem_v)
        cu.wait()
        cv.wait()

        def d_body(d, accs):
            dvec = jnp.full((16,), d, dtype=jnp.int32)
            new = []
            for g in range(NG):
                au = plsc.load_gather(ur, [rows[g], dvec])
                av = plsc.load_gather(vr, [rows[g], dvec])
                new.append(accs[g] + au * av)
            return tuple(new)

        accs = lax.fori_loop(
            0, D, d_body,
            tuple(jnp.zeros((16,), jnp.float32) for _ in range(NG)),
            unroll=4)
        for g in range(NG):
            sc[pl.ds(ci * CH + g * 16, 16)] = accs[g]
        return carry

    lax.fori_loop(0, NCHUNK, chunk_body, 0)
    pltpu.sync_copy(sc, out_hbm.at[pl.ds(base, NE)])


_sc_call = functools.partial(
    pl.kernel,
    out_type=jax.ShapeDtypeStruct((EP,), jnp.float32),
    mesh=plsc.VectorSubcoreMesh(core_axis_name="c", subcore_axis_name="s"),
    compiler_params=pltpu.CompilerParams(
        use_tc_tiling_on_sc=False, needs_layout_passes=False),
    scratch_types=[
        pltpu.VMEM((NE,), jnp.int32),
        pltpu.VMEM((NE,), jnp.int32),
        pltpu.VMEM((CH, D), jnp.float32),
        pltpu.VMEM((CH, D), jnp.float32),
        pltpu.VMEM((NE,), jnp.float32),
        pltpu.SemaphoreType.DMA,
        pltpu.SemaphoreType.DMA,
    ],
)(_body)


def kernel(h, edges):
    u = edges[0].astype(jnp.int32)
    v = edges[1].astype(jnp.int32)
    pad = jnp.zeros((EP - E,), jnp.int32)
    up = jnp.concatenate([u, pad])
    vp = jnp.concatenate([v, pad])
    scores = _sc_call(h, up, vp)
    return scores[:E]


# speedup vs baseline: 2.9879x; 2.9879x over previous
import functools

import jax
import jax.numpy as jnp
from jax import lax
from jax.experimental import pallas as pl
from jax.experimental.pallas import tpu as pltpu
from jax.experimental.pallas import tpu_sc as plsc

D = 256
E = 160000
NW = 32
NE = 5120
EP = NW * NE
CH = 128
NCHUNK = NE // CH
NG = CH // 16


def _body(h_hbm, u_hbm, v_hbm, out_hbm, u_idx, v_idx, ur, vr, sc, sem_u, sem_v):
    wid = lax.axis_index("s") * 2 + lax.axis_index("c")
    base = wid * NE
    pltpu.sync_copy(u_hbm.at[pl.ds(base, NE)], u_idx)
    pltpu.sync_copy(v_hbm.at[pl.ds(base, NE)], v_idx)

    lane = lax.iota(jnp.int32, 16)
    rows = [lane + g * 16 for g in range(NG)]

    def chunk_body(ci, carry):
        cu = pltpu.async_copy(
            h_hbm.at[u_idx.at[pl.ds(ci * CH, CH)]], ur, sem_u)
        cv = pltpu.async_copy(
            h_hbm.at[v_idx.at[pl.ds(ci * CH, CH)]], vr, sem_v)
        cu.wait()
        cv.wait()

        def d_body(d, accs):
            dvec = jnp.bitwise_and(d + lane, D - 1)
            new = []
            for g in range(NG):
                au = plsc.load_gather(ur, [rows[g], dvec])
                av = plsc.load_gather(vr, [rows[g], dvec])
                new.append(accs[g] + au * av)
            return tuple(new)

        accs = lax.fori_loop(
            0, D, d_body,
            tuple(jnp.zeros((16,), jnp.float32) for _ in range(NG)),
            unroll=4)
        for g in range(NG):
            sc[pl.ds(ci * CH + g * 16, 16)] = accs[g]
        return carry

    lax.fori_loop(0, NCHUNK, chunk_body, 0)
    pltpu.sync_copy(sc, out_hbm.at[pl.ds(base, NE)])


_sc_call = functools.partial(
    pl.kernel,
    out_type=jax.ShapeDtypeStruct((EP,), jnp.float32),
    mesh=plsc.VectorSubcoreMesh(core_axis_name="c", subcore_axis_name="s"),
    compiler_params=pltpu.CompilerParams(
        use_tc_tiling_on_sc=False, needs_layout_passes=False),
    scratch_types=[
        pltpu.VMEM((NE,), jnp.int32),
        pltpu.VMEM((NE,), jnp.int32),
        pltpu.VMEM((CH, D), jnp.float32),
        pltpu.VMEM((CH, D), jnp.float32),
        pltpu.VMEM((NE,), jnp.float32),
        pltpu.SemaphoreType.DMA,
        pltpu.SemaphoreType.DMA,
    ],
)(_body)


def kernel(h, edges):
    u = edges[0].astype(jnp.int32)
    v = edges[1].astype(jnp.int32)
    pad = jnp.zeros((EP - E,), jnp.int32)
    up = jnp.concatenate([u, pad])
    vp = jnp.concatenate([v, pad])
    scores = _sc_call(h, up, vp)
    return scores[:E]


# speedup vs baseline: 3.5260x; 1.1801x over previous
import functools

import jax
import jax.numpy as jnp
from jax import lax
from jax.experimental import pallas as pl
from jax.experimental.pallas import tpu as pltpu
from jax.experimental.pallas import tpu_sc as plsc

D = 256
E = 160000
NW = 32
NE = 5120
EP = NW * NE
CH = 80
NCHUNK = NE // CH
NG = CH // 16


def _body(h_hbm, u_hbm, v_hbm, out_hbm,
          u_idx, v_idx, ur0, vr0, ur1, vr1, sc,
          su0, sv0, su1, sv1):
    wid = lax.axis_index("s") * 2 + lax.axis_index("c")
    base = wid * NE
    pltpu.sync_copy(u_hbm.at[pl.ds(base, NE)], u_idx)
    pltpu.sync_copy(v_hbm.at[pl.ds(base, NE)], v_idx)

    ubufs, vbufs = (ur0, ur1), (vr0, vr1)
    usems, vsems = (su0, su1), (sv0, sv1)
    lane = lax.iota(jnp.int32, 16)
    rows = [lane + g * 16 for g in range(NG)]

    def issue(ci, b):
        pltpu.async_copy(
            h_hbm.at[u_idx.at[pl.ds(ci * CH, CH)]], ubufs[b], usems[b])
        pltpu.async_copy(
            h_hbm.at[v_idx.at[pl.ds(ci * CH, CH)]], vbufs[b], vsems[b])

    def wait(b):
        pltpu.make_async_copy(
            h_hbm.at[u_idx.at[pl.ds(0, CH)]], ubufs[b], usems[b]).wait()
        pltpu.make_async_copy(
            h_hbm.at[v_idx.at[pl.ds(0, CH)]], vbufs[b], vsems[b]).wait()

    def compute(ci, b):
        ub, vb = ubufs[b], vbufs[b]

        def acc_body(d, acc):
            dvec = jnp.bitwise_and(d + lane, D - 1)
            new = []
            for g in range(NG):
                au = plsc.load_gather(ub, [rows[g], dvec])
                av = plsc.load_gather(vb, [rows[g], dvec])
                new.append(acc[g] + au * av)
            return tuple(new)

        accs = plsc.parallel_loop(
            0, D, unroll=8,
            carry=tuple(jnp.zeros((16,), jnp.float32) for _ in range(NG)),
        )(acc_body)
        for g in range(NG):
            sc[pl.ds(ci * CH + g * 16, 16)] = accs[g]

    issue(0, 0)

    def pair_body(k, carry):
        ci = 2 * k
        issue(ci + 1, 1)
        wait(0)
        compute(ci, 0)

        @pl.when(k < NCHUNK // 2 - 1)
        def _():
            issue(ci + 2, 0)

        wait(1)
        compute(ci + 1, 1)
        return carry

    lax.fori_loop(0, NCHUNK // 2, pair_body, 0)
    pltpu.sync_copy(sc, out_hbm.at[pl.ds(base, NE)])


_sc_call = functools.partial(
    pl.kernel,
    out_type=jax.ShapeDtypeStruct((EP,), jnp.float32),
    mesh=plsc.VectorSubcoreMesh(core_axis_name="c", subcore_axis_name="s"),
    compiler_params=pltpu.CompilerParams(
        use_tc_tiling_on_sc=False, needs_layout_passes=False),
    scratch_types=[
        pltpu.VMEM((NE,), jnp.int32),
        pltpu.VMEM((NE,), jnp.int32),
        pltpu.VMEM((CH, D), jnp.float32),
        pltpu.VMEM((CH, D), jnp.float32),
        pltpu.VMEM((CH, D), jnp.float32),
        pltpu.VMEM((CH, D), jnp.float32),
        pltpu.VMEM((NE,), jnp.float32),
        pltpu.SemaphoreType.DMA,
        pltpu.SemaphoreType.DMA,
        pltpu.SemaphoreType.DMA,
        pltpu.SemaphoreType.DMA,
    ],
)(_body)


def kernel(h, edges):
    u = edges[0].astype(jnp.int32)
    v = edges[1].astype(jnp.int32)
    pad = jnp.zeros((EP - E,), jnp.int32)
    up = jnp.concatenate([u, pad])
    vp = jnp.concatenate([v, pad])
    scores = _sc_call(h, up, vp)
    return scores[:E]
